# baseline (speedup 1.0000x reference)
import functools

import jax
import jax.numpy as jnp
from jax import lax
from jax.experimental import pallas as pl
from jax.experimental.pallas import tpu as pltpu
from jax.experimental.pallas import tpu_sc as plsc

TEMP_INV = 1.0 / 3.0
TOPK = 10
L = 128
NGSEL = 16
NEG = -3.0e38
BIGI = 2 ** 30


def _tc_pass(part_features, mem_t, rb):
    K, B, D = part_features.shape
    N = mem_t.shape[2]
    ng = N // L
    cw = min(8192, N)

    def body(feat_ref, memt_ref, sim_ref, cand_ref, mnt_ref):
        r = pl.program_id(1)

        @pl.when(r == 0)
        def _():
            for cb in range(N // cw):
                x = memt_ref[0, :, pl.ds(cb * cw, cw)]
                nrm = jnp.sqrt(jnp.sum(x * x, axis=0, keepdims=True))
                mnt_ref[:, pl.ds(cb * cw, cw)] = x / jnp.maximum(nrm, 1e-12)

        feat = feat_ref[0]
        fn = feat / jnp.maximum(
            jnp.sqrt(jnp.sum(feat * feat, axis=1, keepdims=True)), 1e-12)
        gs = []
        for cb in range(N // cw):
            simc = lax.dot_general(
                fn, mnt_ref[:, pl.ds(cb * cw, cw)], (((1,), (0,)), ((), ())),
                preferred_element_type=jnp.float32)
            sim_ref[0, :, pl.ds(cb * cw, cw)] = simc
            gs.append(jnp.max(simc.reshape(rb, cw // L, L), axis=2))
        g = jnp.concatenate(gs, axis=1)
        gid = lax.broadcasted_iota(jnp.int32, (rb, ng), 1)
        lane = lax.broadcasted_iota(jnp.int32, (rb, NGSEL), 1)
        c = jnp.zeros((rb, NGSEL), jnp.int32)
        for t in range(NGSEL):
            m = jnp.max(g, axis=1, keepdims=True)
            pos = jnp.min(jnp.where(g >= m, gid, BIGI), axis=1, keepdims=True)
            c = jnp.where(lane == t, pos, c)
            g = jnp.where(gid == pos, NEG, g)
        cand_ref[0] = c

    return pl.pallas_call(
        body,
        grid=(K, B // rb),
        in_specs=[
            pl.BlockSpec((1, rb, D), lambda k, r: (k, r, 0)),
            pl.BlockSpec((1, D, N), lambda k, r: (k, 0, 0)),
        ],
        out_specs=[
            pl.BlockSpec((1, rb, N), lambda k, r: (k, r, 0)),
            pl.BlockSpec((1, rb, NGSEL), lambda k, r: (k, r, 0)),
        ],
        out_shape=[
            jax.ShapeDtypeStruct((K, B, N), jnp.float32),
            jax.ShapeDtypeStruct((K, B, NGSEL), jnp.int32),
        ],
        scratch_shapes=[pltpu.VMEM((D, N), jnp.float32)],
        compiler_params=pltpu.CompilerParams(
            vmem_limit_bytes=100 * 1024 * 1024),
    )(part_features, mem_t)


def _sc_pass(sim_view, cand_view, rows, n):
    ng = n // L
    nw = 32
    rpw = rows // nw
    mesh = plsc.VectorSubcoreMesh(core_axis_name="c", subcore_axis_name="s",
                                  num_cores=2, num_subcores=16)

    @functools.partial(
        pl.kernel,
        out_type=jax.ShapeDtypeStruct((rows, n), jnp.float32),
        mesh=mesh,
        compiler_params=pltpu.CompilerParams(needs_layout_passes=False),
        scratch_types=[
            pltpu.VMEM((n,), jnp.float32),
            pltpu.VMEM((NGSEL, L), jnp.float32),
            pltpu.VMEM((NGSEL,), jnp.int32),
        ],
    )
    def sck(sim_hbm, cand_hbm, out_hbm, zbuf, gbuf, cbuf):
        wid = lax.axis_index("s") * 2 + lax.axis_index("c")
        base = wid * rpw
        z16 = jnp.zeros((16,), jnp.float32)

        def zb(i, carry):
            zbuf[pl.ds(i * 16, 16)] = z16
            return carry
        lax.fori_loop(0, n // 16, zb, 0)

        iota16 = lax.iota(jnp.int32, 16)

        def row_body(i, carry):
            r = base + i
            pltpu.sync_copy(cand_hbm.at[r], cbuf)
            cvec = cbuf[...]
            gidx = cvec + r * ng
            pltpu.sync_copy(sim_hbm.at[gidx], gbuf)
            av = jnp.full((16,), NEG, jnp.float32)
            ai = jnp.zeros((16,), jnp.int32)
            for t in range(NGSEL):
                bsel = cvec.at[jnp.full((16,), t, jnp.int32)].get(
                    mode="promise_in_bounds") * L

                def sub(j, carry2, t=t, bsel=bsel):
                    av2, ai2 = carry2
                    vals = gbuf[t, pl.ds(j * 16, 16)]
                    cols = bsel + j * 16 + iota16
                    sv, sc = plsc.sort_key_val(vals, cols, descending=True)
                    rv = lax.rev(sv, (0,))
                    ri = lax.rev(sc, (0,))
                    keep = av2 >= rv
                    mv = jnp.where(keep, av2, rv)
                    mi = jnp.where(keep, ai2, ri)
                    nv, ni = plsc.sort_key_val(mv, mi, descending=True)
                    return (nv, ni)

                av, ai = lax.fori_loop(0, L // 16, sub, (av, ai))
            topm = iota16 < TOPK
            e = jnp.where(topm, jnp.exp(av * TEMP_INV), 0.0)
            p = e / jnp.sum(e)
            plsc.store_scatter(zbuf, [ai], p)
            pltpu.sync_copy(zbuf, out_hbm.at[r])
            plsc.store_scatter(zbuf, [ai], z16)
            return carry

        lax.fori_loop(0, rpw, row_body, 0)

    return sck(sim_view, cand_view)


@jax.jit
def kernel(part_features, memory):
    K, B, D = part_features.shape
    N = memory.shape[1]
    sim, cand = _tc_pass(part_features, jnp.swapaxes(memory, 1, 2), rb=32)
    sim_view = sim.reshape(K * B * (N // L), L)
    cand_view = cand.reshape(K * B, NGSEL)
    soft = _sc_pass(sim_view, cand_view, K * B, N)
    return soft.reshape(K, B, N), sim
